# baseline (speedup 1.0000x reference)
import jax
import jax.numpy as jnp
from jax.experimental import pallas as pl


def _pos_kernel(row_ref, col_ref, out_ref):
    row_t = row_ref[...].T
    col_t = col_ref[...].T
    row_part = jnp.broadcast_to(row_t[:, :, None], (384, 32, 32))
    col_part = jnp.broadcast_to(col_t[:, None, :], (384, 32, 32))
    out_ref[0] = jnp.concatenate([row_part, col_part], axis=0)


def kernel(feat, row_embed, col_embed):
    B, C, H, W = feat.shape
    half = row_embed.shape[1]
    out = pl.pallas_call(
        _pos_kernel,
        grid=(B,),
        in_specs=[
            pl.BlockSpec((H, half), lambda b: (0, 0)),
            pl.BlockSpec((W, half), lambda b: (0, 0)),
        ],
        out_specs=pl.BlockSpec((1, C, H, W), lambda b: (b, 0, 0, 0)),
        out_shape=jax.ShapeDtypeStruct((B, C, H, W), jnp.float32),
    )(row_embed[:H], col_embed[:W])
    return out


# speedup vs baseline: 3.0922x; 3.0922x over previous
import jax
import jax.numpy as jnp
from jax.experimental import pallas as pl


def _pos_kernel(row_ref, col_ref, out_ref):
    H, half = row_ref.shape
    W = col_ref.shape[0]
    HW = H * W
    p = jax.lax.broadcasted_iota(jnp.int32, (H, HW), 1)
    i = jax.lax.broadcasted_iota(jnp.int32, (H, HW), 0)
    sel_row = (p // W == i).astype(jnp.float32)
    sel_col = (p % W == i).astype(jnp.float32)
    dn = (((0,), (0,)), ((), ()))
    row_part = jax.lax.dot_general(row_ref[...], sel_row, dn,
                                   preferred_element_type=jnp.float32)
    col_part = jax.lax.dot_general(col_ref[...], sel_col, dn,
                                   preferred_element_type=jnp.float32)
    out_ref[0, :half] = row_part
    out_ref[0, half:] = col_part


def kernel(feat, row_embed, col_embed):
    B, C, H, W = feat.shape
    half = row_embed.shape[1]
    out = pl.pallas_call(
        _pos_kernel,
        grid=(B,),
        in_specs=[
            pl.BlockSpec((H, half), lambda b: (0, 0)),
            pl.BlockSpec((W, half), lambda b: (0, 0)),
        ],
        out_specs=pl.BlockSpec((1, C, H * W), lambda b: (b, 0, 0)),
        out_shape=jax.ShapeDtypeStruct((B, C, H * W), jnp.float32),
    )(row_embed[:H], col_embed[:W])
    return out.reshape(B, C, H, W)


# speedup vs baseline: 3.1200x; 1.0090x over previous
import jax
import jax.numpy as jnp
from jax.experimental import pallas as pl
from jax.experimental.pallas import tpu as pltpu


def _pos_kernel(row_ref, col_ref, out_ref, scratch, sem):
    H, half = row_ref.shape
    W = col_ref.shape[0]
    HW = H * W
    B = out_ref.shape[0]
    p = jax.lax.broadcasted_iota(jnp.int32, (H, HW), 1)
    i = jax.lax.broadcasted_iota(jnp.int32, (H, HW), 0)
    sel_row = (p // W == i).astype(jnp.float32)
    sel_col = (p % W == i).astype(jnp.float32)
    dn = (((0,), (0,)), ((), ()))
    scratch[:half] = jax.lax.dot_general(
        row_ref[...], sel_row, dn,
        preferred_element_type=jnp.float32, precision=jax.lax.Precision.HIGHEST)
    scratch[half:] = jax.lax.dot_general(
        col_ref[...], sel_col, dn,
        preferred_element_type=jnp.float32, precision=jax.lax.Precision.HIGHEST)
    copies = [pltpu.make_async_copy(scratch, out_ref.at[b], sem)
              for b in range(B)]
    for c in copies:
        c.start()
    for c in copies:
        c.wait()


def kernel(feat, row_embed, col_embed):
    B, C, H, W = feat.shape
    half = row_embed.shape[1]
    out = pl.pallas_call(
        _pos_kernel,
        in_specs=[
            pl.BlockSpec(memory_space=pltpu.MemorySpace.VMEM),
            pl.BlockSpec(memory_space=pltpu.MemorySpace.VMEM),
        ],
        out_specs=pl.BlockSpec(memory_space=pltpu.MemorySpace.HBM),
        out_shape=jax.ShapeDtypeStruct((B, C, H * W), jnp.float32),
        scratch_shapes=[
            pltpu.VMEM((C, H * W), jnp.float32),
            pltpu.SemaphoreType.DMA,
        ],
    )(row_embed[:H], col_embed[:W])
    return out.reshape(B, C, H, W)


# speedup vs baseline: 3.1218x; 1.0006x over previous
import jax
import jax.numpy as jnp
from jax.experimental import pallas as pl
from jax.experimental.pallas import tpu as pltpu


def _pos_kernel(row_ref, col_ref, out_ref, scratch, sem):
    H, half = row_ref.shape
    W = col_ref.shape[0]
    HW = H * W
    B = out_ref.shape[0]
    p = jax.lax.broadcasted_iota(jnp.int32, (H, HW), 1)
    i = jax.lax.broadcasted_iota(jnp.int32, (H, HW), 0)
    sel_row = (p // W == i).astype(jnp.float32)
    sel_col = (p % W == i).astype(jnp.float32)
    dn = (((0,), (0,)), ((), ()))
    scratch[:half] = jax.lax.dot_general(
        row_ref[...], sel_row, dn,
        preferred_element_type=jnp.float32, precision=jax.lax.Precision.HIGHEST)
    scratch[half:] = jax.lax.dot_general(
        col_ref[...], sel_col, dn,
        preferred_element_type=jnp.float32, precision=jax.lax.Precision.HIGHEST)
    copies = [pltpu.make_async_copy(scratch, out_ref.at[b], sem.at[b])
              for b in range(B)]
    for c in copies:
        c.start()
    for c in copies:
        c.wait()


def kernel(feat, row_embed, col_embed):
    B, C, H, W = feat.shape
    half = row_embed.shape[1]
    out = pl.pallas_call(
        _pos_kernel,
        in_specs=[
            pl.BlockSpec(memory_space=pltpu.MemorySpace.VMEM),
            pl.BlockSpec(memory_space=pltpu.MemorySpace.VMEM),
        ],
        out_specs=pl.BlockSpec(memory_space=pltpu.MemorySpace.HBM),
        out_shape=jax.ShapeDtypeStruct((B, C, H * W), jnp.float32),
        scratch_shapes=[
            pltpu.VMEM((C, H * W), jnp.float32),
            pltpu.SemaphoreType.DMA((16,)),
        ],
    )(row_embed[:H], col_embed[:W])
    return out.reshape(B, C, H, W)


# speedup vs baseline: 3.1704x; 1.0156x over previous
import jax
import jax.numpy as jnp
from jax.experimental import pallas as pl


_CHUNKS_PER_HALF = 3


def _pos_kernel(row_ref, col_ref, out_ref):
    H, ck = row_ref.shape
    W = col_ref.shape[0]
    B, _, HW = out_ref.shape
    k = pl.program_id(0)
    p = jax.lax.broadcasted_iota(jnp.int32, (H, HW), 1)
    i = jax.lax.broadcasted_iota(jnp.int32, (H, HW), 0)
    sel_row = (p // W == i).astype(jnp.float32)
    sel_col = (p % W == i).astype(jnp.float32)
    is_row = k < _CHUNKS_PER_HALF
    blk = jnp.where(is_row, row_ref[...], col_ref[...])
    sel = jnp.where(is_row, sel_row, sel_col)
    dn = (((0,), (0,)), ((), ()))
    chunk = jax.lax.dot_general(blk, sel, dn,
                                preferred_element_type=jnp.float32,
                                precision=jax.lax.Precision.HIGHEST)
    for b in range(B):
        out_ref[b] = chunk


def kernel(feat, row_embed, col_embed):
    B, C, H, W = feat.shape
    half = row_embed.shape[1]
    n_chunks = 2 * _CHUNKS_PER_HALF
    ck = half // _CHUNKS_PER_HALF
    out = pl.pallas_call(
        _pos_kernel,
        grid=(n_chunks,),
        in_specs=[
            pl.BlockSpec((H, ck), lambda k: (0, jnp.minimum(k, _CHUNKS_PER_HALF - 1))),
            pl.BlockSpec((W, ck), lambda k: (0, jnp.maximum(k - _CHUNKS_PER_HALF, 0))),
        ],
        out_specs=pl.BlockSpec((B, ck, H * W), lambda k: (0, k, 0)),
        out_shape=jax.ShapeDtypeStruct((B, C, H * W), jnp.float32),
    )(row_embed[:H], col_embed[:W])
    return out.reshape(B, C, H, W)
